# baseline (speedup 1.0000x reference)
import functools

import jax
import jax.numpy as jnp
from jax import lax
from jax.experimental import pallas as pl
from jax.experimental.pallas import tpu as pltpu
from jax.experimental.pallas import tpu_sc as plsc

B = 4096
S = 200
D = 64
NC, NS = 2, 16
NW = NC * NS
BT = B // NW
SC_CHUNK = 2
NCHUNKS = S // SC_CHUNK
NBUF = 2


def _emb_body(ids_hbm, table_hbm, pos_hbm, time_hbm, out_hbm, time_out,
              pos_v, ib0, ib1, rb0, rb1, tb0, tb1,
              gsem0, gsem1, ssem0, ssem1):
    ibs = [ib0, ib1]
    rbs = [rb0, rb1]
    tbs = [tb0, tb1]
    gsems = [gsem0, gsem1]
    ssems = [ssem0, ssem1]

    wid = lax.axis_index("s") * NC + lax.axis_index("c")
    bt0 = wid * BT

    @pl.when(wid == 0)
    def _():
        pltpu.sync_copy(time_hbm, time_out)

    pltpu.sync_copy(pos_hbm.at[pl.ds(0, S)], pos_v)

    iota = lax.iota(jnp.int32, 16)
    ibt = [iota + 16 * g for g in range(8)]

    def fire(c, ib, rb, gsem):
        pltpu.sync_copy(
            ids_hbm.at[pl.ds(c * SC_CHUNK, SC_CHUNK), pl.ds(bt0, BT)], ib)
        for sl in range(SC_CHUNK):
            pltpu.async_copy(table_hbm.at[ib.at[sl]], rb.at[sl], gsem)

    def drain(ib, rb, gsem):
        for sl in range(SC_CHUNK):
            pltpu.make_async_copy(table_hbm.at[ib.at[sl]], rb.at[sl],
                                  gsem).wait()

    def store(c, tb, ssem):
        pltpu.async_copy(
            tb, out_hbm.at[pl.ds(c * SC_CHUNK, SC_CHUNK), slice(None),
                           pl.ds(bt0, BT)], ssem)

    def wait_store(tb, ssem):
        pltpu.make_async_copy(
            tb, out_hbm.at[pl.ds(0, SC_CHUNK), slice(None), pl.ds(0, BT)],
            ssem).wait()

    def transpose_add(c, rb, tb):
        s0 = c * SC_CHUNK

        def body(cc, carry):
            ccv = jnp.full((16,), cc, jnp.int32)
            for sl in range(SC_CHUNK):
                slv = jnp.full((16,), sl, jnp.int32)
                sv = jnp.full((16,), s0 + sl, jnp.int32)
                posb = plsc.load_gather(pos_v, [sv, ccv])
                for g in range(8):
                    vals = plsc.load_gather(rb, [slv, ibt[g], ccv])
                    tb[sl, cc, pl.ds(16 * g, 16)] = vals + posb
            return carry

        lax.fori_loop(0, D, body, 0)

    for b in range(NBUF):
        fire(b, ibs[b], rbs[b], gsems[b])

    def outer(i, carry):
        for b in range(NBUF):
            c = i * NBUF + b
            drain(ibs[b], rbs[b], gsems[b])

            @pl.when(c >= NBUF)
            def _():
                wait_store(tbs[b], ssems[b])

            transpose_add(c, rbs[b], tbs[b])
            store(c, tbs[b], ssems[b])

            @pl.when(c + NBUF < NCHUNKS)
            def _():
                fire(c + NBUF, ibs[b], rbs[b], gsems[b])
        return carry

    lax.fori_loop(0, NCHUNKS // NBUF, outer, 0)

    for b in range(NBUF):
        wait_store(tbs[b], ssems[b])


@jax.jit
def _emb_lookup(ids_t, W_word, W_pos, W_time):
    mesh = plsc.VectorSubcoreMesh(core_axis_name="c", subcore_axis_name="s")
    kern = functools.partial(
        pl.kernel,
        mesh=mesh,
        compiler_params=pltpu.CompilerParams(
            use_tc_tiling_on_sc=False, needs_layout_passes=False),
        out_type=[
            jax.ShapeDtypeStruct((S, D, B), jnp.float32),
            jax.ShapeDtypeStruct(W_time.shape, jnp.float32),
        ],
        scratch_types=(
            [pltpu.VMEM((S, D), jnp.float32)]
            + [pltpu.VMEM((SC_CHUNK, BT), jnp.int32) for _ in range(NBUF)]
            + [pltpu.VMEM((SC_CHUNK, BT, D), jnp.float32) for _ in range(NBUF)]
            + [pltpu.VMEM((SC_CHUNK, D, BT), jnp.float32) for _ in range(NBUF)]
            + [pltpu.SemaphoreType.DMA for _ in range(2 * NBUF)]
        ),
    )(_emb_body)
    return kern(ids_t, W_word, W_pos, W_time)


def kernel(input_ids, W_word, W_pos, W_time):
    ids_t = input_ids.T
    out_t, time_emb = _emb_lookup(ids_t, W_word, W_pos, W_time)
    return out_t.transpose(2, 0, 1), time_emb[None]


# speedup vs baseline: 1.1119x; 1.1119x over previous
import functools

import jax
import jax.numpy as jnp
from jax import lax
from jax.experimental import pallas as pl
from jax.experimental.pallas import tpu as pltpu
from jax.experimental.pallas import tpu_sc as plsc

B = 4096
S = 200
D = 64
NC, NS = 2, 16
NW = NC * NS
BT = B // NW
SC_CHUNK = 2
NCHUNKS = S // SC_CHUNK
NBUF = 2


def _emb_body(ids_hbm, table_hbm, pos_hbm, time_hbm, out_hbm, time_out,
              pos_v, ib0, ib1, rb0, rb1, tb0, tb1,
              gsem0, gsem1, ssem0, ssem1):
    ibs = [ib0, ib1]
    rbs = [rb0, rb1]
    tbs = [tb0, tb1]
    gsems = [gsem0, gsem1]
    ssems = [ssem0, ssem1]

    wid = lax.axis_index("s") * NC + lax.axis_index("c")
    bt0 = wid * BT

    @pl.when(wid == 0)
    def _():
        pltpu.sync_copy(time_hbm, time_out)

    pltpu.sync_copy(pos_hbm.at[pl.ds(0, S)], pos_v)

    iota = lax.iota(jnp.int32, 16)
    qidx = [iota + 16 * q for q in range(4)]
    slidx = [jnp.full((16,), sl, jnp.int32) for sl in range(SC_CHUNK)]

    def fire(c, ib, rb, gsem):
        pltpu.sync_copy(
            ids_hbm.at[pl.ds(c * SC_CHUNK, SC_CHUNK), pl.ds(bt0, BT)], ib)
        for sl in range(SC_CHUNK):
            pltpu.async_copy(table_hbm.at[ib.at[sl]], rb.at[sl], gsem)

    def drain(ib, rb, gsem):
        for sl in range(SC_CHUNK):
            pltpu.make_async_copy(table_hbm.at[ib.at[sl]], rb.at[sl],
                                  gsem).wait()

    def store(c, tb, ssem):
        pltpu.async_copy(
            tb, out_hbm.at[pl.ds(c * SC_CHUNK, SC_CHUNK), slice(None),
                           pl.ds(bt0, BT)], ssem)

    def wait_store(tb, ssem):
        pltpu.make_async_copy(
            tb, out_hbm.at[pl.ds(0, SC_CHUNK), slice(None), pl.ds(0, BT)],
            ssem).wait()

    def transpose_add(c, rb, tb):
        s0 = c * SC_CHUNK
        pos = [[pos_v[s0 + sl, pl.ds(16 * q, 16)] for q in range(4)]
               for sl in range(SC_CHUNK)]

        def body(r, carry):
            rv = jnp.full((16,), r, jnp.int32)
            for sl in range(SC_CHUNK):
                for q in range(4):
                    vals = rb[sl, r, pl.ds(16 * q, 16)] + pos[sl][q]
                    plsc.store_scatter(tb, [slidx[sl], qidx[q], rv], vals)
            return carry

        lax.fori_loop(0, BT, body, 0)

    for b in range(NBUF):
        fire(b, ibs[b], rbs[b], gsems[b])

    def outer(i, carry):
        for b in range(NBUF):
            c = i * NBUF + b
            drain(ibs[b], rbs[b], gsems[b])

            @pl.when(c >= NBUF)
            def _():
                wait_store(tbs[b], ssems[b])

            transpose_add(c, rbs[b], tbs[b])
            store(c, tbs[b], ssems[b])

            @pl.when(c + NBUF < NCHUNKS)
            def _():
                fire(c + NBUF, ibs[b], rbs[b], gsems[b])
        return carry

    lax.fori_loop(0, NCHUNKS // NBUF, outer, 0)

    for b in range(NBUF):
        wait_store(tbs[b], ssems[b])


@jax.jit
def _emb_lookup(ids_t, W_word, W_pos, W_time):
    mesh = plsc.VectorSubcoreMesh(core_axis_name="c", subcore_axis_name="s")
    kern = functools.partial(
        pl.kernel,
        mesh=mesh,
        compiler_params=pltpu.CompilerParams(
            use_tc_tiling_on_sc=False, needs_layout_passes=False),
        out_type=[
            jax.ShapeDtypeStruct((S, D, B), jnp.float32),
            jax.ShapeDtypeStruct(W_time.shape, jnp.float32),
        ],
        scratch_types=(
            [pltpu.VMEM((S, D), jnp.float32)]
            + [pltpu.VMEM((SC_CHUNK, BT), jnp.int32) for _ in range(NBUF)]
            + [pltpu.VMEM((SC_CHUNK, BT, D), jnp.float32) for _ in range(NBUF)]
            + [pltpu.VMEM((SC_CHUNK, D, BT), jnp.float32) for _ in range(NBUF)]
            + [pltpu.SemaphoreType.DMA for _ in range(2 * NBUF)]
        ),
    )(_emb_body)
    return kern(ids_t, W_word, W_pos, W_time)


def kernel(input_ids, W_word, W_pos, W_time):
    ids_t = input_ids.T
    out_t, time_emb = _emb_lookup(ids_t, W_word, W_pos, W_time)
    return out_t.transpose(2, 0, 1), time_emb[None]


# speedup vs baseline: 1.7738x; 1.5953x over previous
import functools

import jax
import jax.numpy as jnp
from jax import lax
from jax.experimental import pallas as pl
from jax.experimental.pallas import tpu as pltpu
from jax.experimental.pallas import tpu_sc as plsc

B = 4096
S = 200
D = 64
N = B * S
NC, NS = 2, 16
NW = NC * NS
ROWS_PER_W = N // NW
GROUP = 100
GROUPS_PER_CHUNK = 4
CHUNK = GROUP * GROUPS_PER_CHUNK
SEQ_PER_CHUNK = CHUNK // S
NCHUNKS = ROWS_PER_W // CHUNK
NBUF = 4
NITER = NCHUNKS // NBUF
NCHUNKS_TOTAL = N // CHUNK


def _emb_body(ids_hbm, table_hbm, pos_hbm, time_hbm, out_hbm, time_out,
              pos_v,
              ib0, ib1, ib2, ib3,
              rb0, rb1, rb2, rb3,
              gsem0, gsem1, gsem2, gsem3,
              ssem0, ssem1, ssem2, ssem3):
    ibs = [ib0, ib1, ib2, ib3]
    rbs = [rb0, rb1, rb2, rb3]
    gsems = [gsem0, gsem1, gsem2, gsem3]
    ssems = [ssem0, ssem1, ssem2, ssem3]

    wid = lax.axis_index("s") * NC + lax.axis_index("c")
    row0 = wid * ROWS_PER_W
    chunk0 = wid * NCHUNKS

    @pl.when(wid == 0)
    def _():
        pltpu.sync_copy(time_hbm, time_out)

    pltpu.sync_copy(pos_hbm.at[pl.ds(0, S)], pos_v)

    def fire(c, ib, rb, gsem):
        pltpu.sync_copy(ids_hbm.at[chunk0 + c], ib)
        for j in range(GROUPS_PER_CHUNK):
            pltpu.async_copy(table_hbm.at[ib.at[j]],
                             rb.at[pl.ds(j * GROUP, GROUP)], gsem)

    def drain_gathers(rb, gsem):
        pltpu.make_async_copy(out_hbm.at[pl.ds(0, CHUNK)], rb, gsem).wait()

    def wait_store(rb, ssem):
        pltpu.make_async_copy(rb, out_hbm.at[pl.ds(0, CHUNK)], ssem).wait()

    def add_pos(rb):
        def body(s, carry):
            p = [pos_v[s, pl.ds(16 * q, 16)] for q in range(4)]
            for t in range(SEQ_PER_CHUNK):
                r = t * S + s
                for q in range(4):
                    rb[r, pl.ds(16 * q, 16)] = rb[r, pl.ds(16 * q, 16)] + p[q]
            return carry
        lax.fori_loop(0, S, body, 0)

    for b in range(NBUF):
        fire(b, ibs[b], rbs[b], gsems[b])

    def outer(i, carry):
        for b in range(NBUF):
            c = i * NBUF + b
            drain_gathers(rbs[b], gsems[b])
            add_pos(rbs[b])
            pltpu.async_copy(rbs[b], out_hbm.at[pl.ds(row0 + c * CHUNK, CHUNK)],
                             ssems[b])
            pb = (b - 1) % NBUF
            cc = c + NBUF - 1
            @pl.when((c >= 1) & (cc < NCHUNKS))
            def _():
                wait_store(rbs[pb], ssems[pb])
                fire(cc, ibs[pb], rbs[pb], gsems[pb])
        return carry

    lax.fori_loop(0, NITER, outer, 0)

    for b in range(NBUF):
        wait_store(rbs[b], ssems[b])


@functools.partial(jax.jit, static_argnums=())
def _emb_lookup(ids2d, W_word, W_pos, W_time):
    mesh = plsc.VectorSubcoreMesh(core_axis_name="c", subcore_axis_name="s")
    kern = functools.partial(
        pl.kernel,
        mesh=mesh,
        compiler_params=pltpu.CompilerParams(use_tc_tiling_on_sc=False),
        out_type=[
            jax.ShapeDtypeStruct((N, D), jnp.float32),
            jax.ShapeDtypeStruct(W_time.shape, jnp.float32),
        ],
        scratch_types=(
            [pltpu.VMEM((S, D), jnp.float32)]
            + [pltpu.VMEM((GROUPS_PER_CHUNK, GROUP), jnp.int32) for _ in range(NBUF)]
            + [pltpu.VMEM((CHUNK, D), jnp.float32) for _ in range(NBUF)]
            + [pltpu.SemaphoreType.DMA for _ in range(2 * NBUF)]
        ),
    )(_emb_body)
    return kern(ids2d, W_word, W_pos, W_time)


def kernel(input_ids, W_word, W_pos, W_time):
    ids3d = input_ids.reshape(NCHUNKS_TOTAL, GROUPS_PER_CHUNK, GROUP)
    out_flat, time_emb = _emb_lookup(ids3d, W_word, W_pos, W_time)
    return out_flat.reshape(B, S, D), time_emb[None]
